# baseline (speedup 1.0000x reference)
import functools

import jax
import jax.numpy as jnp
from jax import lax
from jax.experimental import pallas as pl
from jax.experimental.pallas import tpu as pltpu
from jax.experimental.pallas import tpu_sc as plsc

N = 100000
E = 3200000
G = 64
F = 21
DP = 32
HL = 16
NC = 2
NT = 16
CK = 128
M = 8
CHUNKS = E // CK
CPT = 1568
MPT0, MPT1 = CPT // M, (CHUNKS - 15 * CPT) // M
DCPT = 784
DMPT0, DMPT1 = DCPT // M, (CHUNKS - 31 * DCPT) // M
RSL = 11
NPAD = 100096
RPT = NPAD // NT
ZR = 136

ROWS = 5888
NBLK = NPAD // ROWS
FP = 128
PKB = ROWS * DP // 128
PKH = ROWS * HL // 128


def _bcast_lane(vec, i):
    idx = jnp.full((16, 1), i, jnp.int32)
    dn = lax.GatherDimensionNumbers(offset_dims=(), collapsed_slice_dims=(0,),
                                    start_index_map=(0,))
    return lax.gather(vec, idx, dn, slice_sizes=(1,),
                      mode=lax.GatherScatterMode.PROMISE_IN_BOUNDS)


def _sc_compiler_params():
    return pltpu.CompilerParams(needs_layout_passes=False,
                                use_tc_tiling_on_sc=False)


def _mesh():
    return plsc.VectorSubcoreMesh(core_axis_name="c", subcore_axis_name="s",
                                  num_cores=NC, num_subcores=NT)


def _zero_acc(zbuf, acc, t):
    for i in range(ZR):
        zbuf[i, :] = jnp.zeros((HL,), jnp.float32)

    @pl.loop(0, RPT // ZR)
    def _(k):
        pltpu.sync_copy(zbuf, acc.at[pl.ds(t * RPT + k * ZR, ZR)])


def _deg_body(dst_hbm, ew_hbm, deg_hbm, dstv, ewv, valsv, zbuf, acc, isem, ssem):
    c = lax.axis_index("c")
    t = lax.axis_index("s")
    _zero_acc(zbuf, acc, t)
    plsc.subcore_barrier()

    iota = lax.iota(jnp.int32, HL)
    onehots = [jnp.where(iota == i, 1.0, 0.0).astype(jnp.float32) for i in range(HL)]
    w = c * NT + t
    base = w * DCPT
    nm = jnp.where(w < 31, DMPT0, DMPT1)

    def fetch(m, mb):
        ck = base + m * M
        pltpu.async_copy(dst_hbm.at[pl.ds(ck, M)], dstv.at[mb], isem)
        pltpu.async_copy(ew_hbm.at[pl.ds(ck, M)], ewv.at[mb], isem)

    def fetch_wait(m, mb):
        ck = base + m * M
        pltpu.make_async_copy(dst_hbm.at[pl.ds(ck, M)], dstv.at[mb], isem).wait()
        pltpu.make_async_copy(ew_hbm.at[pl.ds(ck, M)], ewv.at[mb], isem).wait()

    fetch(0, 0)

    @pl.loop(0, nm)
    def _(m):
        mb = lax.rem(m, 2)
        pb = lax.rem(m + 1, 2)
        fetch_wait(m, mb)

        @pl.loop(0, M)
        def _(j):
            @pl.when(m > 0)
            def _():
                pltpu.make_async_copy(valsv.at[pl.ds(j * CK, CK)],
                                      acc.at[dstv.at[pb].at[j]], ssem).wait()

        @pl.when(m + 1 < nm)
        def _():
            fetch(m + 1, pb)

        @pl.loop(0, M)
        def _(j):
            for g in range(8):
                ew16 = ewv[mb, j, pl.ds(g * 16, 16)]
                for i in range(16):
                    valsv[j * CK + g * 16 + i, :] = ew16 * onehots[i]
            pltpu.async_copy(valsv.at[pl.ds(j * CK, CK)],
                             acc.at[dstv.at[mb].at[j]], ssem, add=True)

    lb = lax.rem(nm - 1, 2)

    @pl.loop(0, M)
    def _(j):
        pltpu.make_async_copy(valsv.at[pl.ds(j * CK, CK)],
                              acc.at[dstv.at[lb].at[j]], ssem).wait()
    plsc.subcore_barrier()
    pltpu.sync_copy(acc.at[pl.ds(t * RPT, RPT)],
                    deg_hbm.at[c].at[pl.ds(t * RPT, RPT)])


def _sc_deg(dst2, ew2):
    body = functools.partial(
        pl.kernel,
        out_type=jax.ShapeDtypeStruct((NC, NPAD, HL), jnp.float32),
        mesh=_mesh(),
        compiler_params=_sc_compiler_params(),
        scratch_types=[
            pltpu.VMEM((2, M, CK), jnp.int32),
            pltpu.VMEM((2, M, CK), jnp.float32),
            pltpu.VMEM((M * CK, HL), jnp.float32),
            pltpu.VMEM((ZR, HL), jnp.float32),
            pltpu.VMEM_SHARED((NPAD, HL), jnp.float32),
            pltpu.SemaphoreType.DMA,
            pltpu.SemaphoreType.DMA,
        ],
    )(_deg_body)
    return body(dst2, ew2)


def _agg_body(src_hbm, dst_hbm, ew_hbm, tab_hbm, out_hbm,
              srcv, dstv, ewv, rowsv, zbuf, acc, isem, gsem, ssem):
    c = lax.axis_index("c")
    t = lax.axis_index("s")
    _zero_acc(zbuf, acc, t)
    plsc.subcore_barrier()

    base = t * CPT
    nm = jnp.where(t < 15, MPT0, MPT1)

    def fetch(m, mb):
        ck = base + m * M
        pltpu.async_copy(src_hbm.at[c].at[pl.ds(ck, M)], srcv.at[mb], isem)
        pltpu.async_copy(dst_hbm.at[pl.ds(ck, M)], dstv.at[mb], isem)
        pltpu.async_copy(ew_hbm.at[pl.ds(ck, M)], ewv.at[mb], isem)

    def fetch_wait(m, mb):
        ck = base + m * M
        pltpu.make_async_copy(src_hbm.at[c].at[pl.ds(ck, M)], srcv.at[mb], isem).wait()
        pltpu.make_async_copy(dst_hbm.at[pl.ds(ck, M)], dstv.at[mb], isem).wait()
        pltpu.make_async_copy(ew_hbm.at[pl.ds(ck, M)], ewv.at[mb], isem).wait()

    fetch(0, 0)

    @pl.loop(0, nm)
    def _(m):
        mb = lax.rem(m, 2)
        pb = lax.rem(m + 1, 2)
        fetch_wait(m, mb)
        @pl.loop(0, M)
        def _(j):
            g = m * M + j
            slot = lax.rem(g, RSL)

            @pl.when(g >= RSL)
            def _():
                pltpu.make_async_copy(rowsv.at[pl.ds(slot * CK, CK)],
                                      acc.at[dstv.at[mb].at[j]], ssem).wait()
            pltpu.async_copy(tab_hbm.at[srcv.at[mb].at[j]],
                             rowsv.at[pl.ds(slot * CK, CK)], gsem)

        @pl.when(m + 1 < nm)
        def _():
            fetch(m + 1, pb)

        @pl.loop(0, M)
        def _(j):
            g = m * M + j
            slot = lax.rem(g, RSL)
            pltpu.make_async_copy(tab_hbm.at[srcv.at[mb].at[j]],
                                  rowsv.at[pl.ds(slot * CK, CK)], gsem).wait()
            for gg in range(8):
                ew16 = ewv[mb, j, pl.ds(gg * 16, 16)]
                for i in range(16):
                    b = _bcast_lane(ew16, i)
                    e0 = gg * 16 + i
                    rowsv[slot * CK + e0, :] = rowsv[slot * CK + e0, :] * b
            pltpu.async_copy(rowsv.at[pl.ds(slot * CK, CK)],
                             acc.at[dstv.at[mb].at[j]], ssem, add=True)

    lb = lax.rem(nm - 1, 2)

    @pl.loop(0, RSL)
    def _(j):
        pltpu.make_async_copy(rowsv.at[pl.ds(j * CK, CK)],
                              acc.at[dstv.at[lb].at[0]], ssem).wait()
    plsc.subcore_barrier()
    pltpu.sync_copy(acc.at[pl.ds(t * RPT, RPT)],
                    out_hbm.at[c].at[pl.ds(t * RPT, RPT)])


def _sc_agg(src2, dst2, ew2, tab2):
    body = functools.partial(
        pl.kernel,
        out_type=jax.ShapeDtypeStruct((NC, NPAD, HL), jnp.float32),
        mesh=_mesh(),
        compiler_params=_sc_compiler_params(),
        scratch_types=[
            pltpu.VMEM((2, M, CK), jnp.int32),
            pltpu.VMEM((2, M, CK), jnp.int32),
            pltpu.VMEM((2, M, CK), jnp.float32),
            pltpu.VMEM((RSL * CK, HL), jnp.float32),
            pltpu.VMEM((ZR, HL), jnp.float32),
            pltpu.VMEM_SHARED((NPAD, HL), jnp.float32),
            pltpu.SemaphoreType.DMA,
            pltpu.SemaphoreType.DMA,
            pltpu.SemaphoreType.DMA,
        ],
    )(_agg_body)
    return body(src2, dst2, ew2, tab2)


PKR = NPAD * HL // 128
PKH = ROWS * HL // 128


def _pre_kernel(deg3_ref, x_ref, K_ref, D_ref, dis_ref, hs_ref):
    d = deg3_ref[0] + deg3_ref[1]
    deg = jnp.dot(d, K_ref[...], preferred_element_type=jnp.float32) + 1.0
    dis = jnp.where(deg > 0, lax.rsqrt(deg), 0.0)
    dis_ref[...] = dis
    x0 = x_ref[0]
    x1 = x_ref[1]
    for cc in range(NC):
        h = (jnp.dot(x0, D_ref[0, cc], preferred_element_type=jnp.float32)
             + jnp.dot(x1, D_ref[1, cc], preferred_element_type=jnp.float32))
        hs_ref[cc] = h * dis


def _tc_pre(deg3p, xp, K, D):
    return pl.pallas_call(
        _pre_kernel,
        grid=(NBLK,),
        in_specs=[
            pl.BlockSpec((NC, PKH, 128), lambda i: (0, i, 0)),
            pl.BlockSpec((NC, PKH, 128), lambda i: (0, i, 0)),
            pl.BlockSpec((128, 128), lambda i: (0, 0)),
            pl.BlockSpec((NC, NC, 128, 128), lambda i: (0, 0, 0, 0)),
        ],
        out_specs=[
            pl.BlockSpec((PKH, 128), lambda i: (i, 0)),
            pl.BlockSpec((NC, PKH, 128), lambda i: (0, i, 0)),
        ],
        out_shape=[
            jax.ShapeDtypeStruct((PKR, 128), jnp.float32),
            jax.ShapeDtypeStruct((NC, PKR, 128), jnp.float32),
        ],
    )(deg3p, xp, K, D)


def _mid_kernel(acc3_ref, hs3_ref, dis_ref, D_ref, b_ref, out_ref):
    dis = dis_ref[...]
    x0 = dis * (acc3_ref[0] + hs3_ref[0]) + b_ref[0]
    x1 = dis * (acc3_ref[1] + hs3_ref[1]) + b_ref[1]
    for cc in range(NC):
        h = (jnp.dot(x0, D_ref[0, cc], preferred_element_type=jnp.float32)
             + jnp.dot(x1, D_ref[1, cc], preferred_element_type=jnp.float32))
        out_ref[cc] = h * dis


def _tc_mid(acc3p, hs3p, disp, D, bpk):
    return pl.pallas_call(
        _mid_kernel,
        grid=(NBLK,),
        in_specs=[
            pl.BlockSpec((NC, PKH, 128), lambda i: (0, i, 0)),
            pl.BlockSpec((NC, PKH, 128), lambda i: (0, i, 0)),
            pl.BlockSpec((PKH, 128), lambda i: (i, 0)),
            pl.BlockSpec((NC, NC, 128, 128), lambda i: (0, 0, 0, 0)),
            pl.BlockSpec((NC, 1, 128), lambda i: (0, 0, 0)),
        ],
        out_specs=pl.BlockSpec((NC, PKH, 128), lambda i: (0, i, 0)),
        out_shape=jax.ShapeDtypeStruct((NC, PKR, 128), jnp.float32),
    )(acc3p, hs3p, disp, D, bpk)


def _head_kernel(acc3_ref, hs3_ref, dis_ref, b_ref, batch_ref, pW_ref, pb_ref,
                 f1W_ref, f1b_ref, f2W_ref, f2b_ref, oW_ref, ob_ref,
                 out_ref, acc_ref):
    i = pl.program_id(0)

    @pl.when(i == 0)
    def _():
        acc_ref[...] = jnp.zeros_like(acc_ref)

    dis = dis_ref[...]
    x0 = jnp.maximum(dis * (acc3_ref[0] + hs3_ref[0]) + b_ref[0], 0.0)
    x1 = jnp.maximum(dis * (acc3_ref[1] + hs3_ref[1]) + b_ref[1], 0.0)
    lane = lax.broadcasted_iota(jnp.int32, (PKH, 128), 1)
    x1 = jnp.where(lax.rem(lane, 16) == 15, 1.0, x1)
    gid = lax.broadcasted_iota(jnp.int32, (G, PKH), 0)
    pooled = jnp.zeros((G, DP), jnp.float32)
    for k in range(8):
        bk = batch_ref[0, k]
        mk = (gid == bk[None, :]).astype(jnp.float32)
        s0 = jnp.dot(mk, x0[:, 16 * k:16 * k + 16],
                     preferred_element_type=jnp.float32)
        s1 = jnp.dot(mk, x1[:, 16 * k:16 * k + 16],
                     preferred_element_type=jnp.float32)
        pooled += jnp.concatenate([s0, s1], axis=1)
    acc_ref[...] += pooled

    @pl.when(i == NBLK - 1)
    def _():
        acc = acc_ref[...]
        cnt = jnp.maximum(acc[:, DP - 1], 1.0)
        mean = acc / cnt[:, None]

        def leaky(v):
            return jnp.where(v > 0, v, 0.01 * v)
        z = leaky(jnp.dot(mean, pW_ref[...], preferred_element_type=jnp.float32)
                  + pb_ref[...])
        z = leaky(jnp.dot(z, f1W_ref[...], preferred_element_type=jnp.float32)
                  + f1b_ref[...])
        z = leaky(jnp.dot(z, f2W_ref[...], preferred_element_type=jnp.float32)
                  + f2b_ref[...])
        logits = jnp.dot(z, oW_ref[...], preferred_element_type=jnp.float32) + ob_ref[...]
        lane2 = lax.broadcasted_iota(jnp.int32, logits.shape, 1)
        valid = lane2 < 5
        m = jnp.max(jnp.where(valid, logits, -jnp.inf), axis=1, keepdims=True)
        e = jnp.where(valid, jnp.exp(logits - m), 0.0)
        out_ref[...] = e / jnp.sum(e, axis=1, keepdims=True)


def _tc_head(acc3p, hs3p, disp, b3pk, batchS, pWp, pbp, f1W, f1bp, f2Wp, f2bp,
             oWp, obp):
    full = lambda shape: pl.BlockSpec(shape, lambda i: (0,) * len(shape))
    return pl.pallas_call(
        _head_kernel,
        grid=(NBLK,),
        in_specs=[
            pl.BlockSpec((NC, PKH, 128), lambda i: (0, i, 0)),
            pl.BlockSpec((NC, PKH, 128), lambda i: (0, i, 0)),
            pl.BlockSpec((PKH, 128), lambda i: (i, 0)),
            full((NC, 1, 128)),
            pl.BlockSpec((1, 8, PKH), lambda i: (i, 0, 0)),
            full((DP, FP)), full((1, FP)),
            full((FP, 256)), full((1, 256)),
            full((256, FP)), full((1, FP)),
            full((FP, FP)), full((1, FP)),
        ],
        out_specs=pl.BlockSpec((G, FP), lambda i: (0, 0)),
        out_shape=jax.ShapeDtypeStruct((G, FP), jnp.float32),
        scratch_shapes=[pltpu.VMEM((G, DP), jnp.float32)],
    )(acc3p, hs3p, disp, b3pk, batchS, pWp, pbp, f1W, f1bp, f2Wp, f2bp, oWp, obp)


def _blockdiag(Wp):
    eye8 = jnp.eye(8, dtype=jnp.float32)
    blocks = [[jnp.kron(eye8, Wp[16 * c:16 * c + 16, 16 * cc:16 * cc + 16])
               for cc in range(NC)] for c in range(NC)]
    return jnp.stack([jnp.stack(r) for r in blocks])


def _bias_pk(bp):
    return jnp.stack([jnp.tile(bp[0, 16 * c:16 * c + 16], 8)[None, :]
                      for c in range(NC)])


def kernel(prot_x, prot_edge_index, prot_dist, prot_batch, W1, b1, W2, b2, W3,
           b3, pW, pb, f1W, f1b, f2W, f2b, oW, ob):
    src2 = prot_edge_index[0].reshape(CHUNKS, CK)
    srcs2 = jnp.stack([src2, src2 + NPAD])
    dst2 = prot_edge_index[1].reshape(CHUNKS, CK)
    ew2 = prot_dist.reshape(CHUNKS, CK)

    x32 = jnp.pad(prot_x, ((0, NPAD - N), (0, DP - F)))
    xp = jnp.stack([x32[:, :HL], x32[:, HL:]]).reshape(NC, PKR, 128)
    K = jnp.kron(jnp.eye(8, dtype=jnp.float32), jnp.ones((HL, HL), jnp.float32))
    W1p = jnp.pad(W1, ((0, DP - F), (0, DP - F)))
    W2p = jnp.pad(W2, ((0, DP - F), (0, DP - F)))
    W3p = jnp.pad(W3, ((0, DP - F), (0, DP - F)))
    D1, D2, D3 = _blockdiag(W1p), _blockdiag(W2p), _blockdiag(W3p)
    b1k = _bias_pk(jnp.pad(b1, (0, DP - F))[None, :])
    b2k = _bias_pk(jnp.pad(b2, (0, DP - F))[None, :])
    b3k = _bias_pk(jnp.pad(b3, (0, DP - F))[None, :])

    pk = lambda a3: a3.reshape(NC, PKR, 128)
    tab = lambda h: h.reshape(NC * NPAD, HL)
    deg3 = _sc_deg(dst2, ew2)
    disp, hs3p = _tc_pre(pk(deg3), xp, K, D1)
    acc3 = _sc_agg(srcs2, dst2, ew2, tab(hs3p))
    hs3p = _tc_mid(pk(acc3), hs3p, disp, D2, b1k)
    acc3 = _sc_agg(srcs2, dst2, ew2, tab(hs3p))
    hs3p = _tc_mid(pk(acc3), hs3p, disp, D3, b2k)
    acc3 = _sc_agg(srcs2, dst2, ew2, tab(hs3p))

    batchS = jnp.pad(prot_batch, (0, NPAD - N), constant_values=G) \
        .reshape(NBLK, PKH, 8).transpose(0, 2, 1)
    pWp = jnp.pad(pW, ((0, DP - F), (0, 0)))
    pbp = pb[None, :]
    f1bp = f1b[None, :]
    f2Wp = jnp.pad(f2W, ((0, 0), (0, FP - 64)))
    f2bp = jnp.pad(f2b, (0, FP - 64))[None, :]
    oWp = jnp.pad(oW, ((0, FP - 64), (0, FP - 5)))
    obp = jnp.pad(ob, (0, FP - 5))[None, :]
    out = _tc_head(pk(acc3), hs3p, disp, b3k, batchS, pWp, pbp, f1W, f1bp,
                   f2Wp, f2bp, oWp, obp)
    return out[:, :5]


# speedup vs baseline: 1.0036x; 1.0036x over previous
import functools

import jax
import jax.numpy as jnp
from jax import lax
from jax.experimental import pallas as pl
from jax.experimental.pallas import tpu as pltpu
from jax.experimental.pallas import tpu_sc as plsc

N = 100000
E = 3200000
G = 64
F = 21
DP = 32
HL = 16
NC = 2
NT = 16
CK = 128
M = 8
CHUNKS = E // CK
CPT = 1568
MPT0, MPT1 = CPT // M, (CHUNKS - 15 * CPT) // M
DCPT = 784
DMPT0, DMPT1 = DCPT // M, (CHUNKS - 31 * DCPT) // M
NPAD = 100096
RPT = NPAD // NT
ZR = 136

ROWS = 5888
NBLK = NPAD // ROWS
FP = 128
PKB = ROWS * DP // 128
PKH = ROWS * HL // 128


def _bcast_lane(vec, i):
    idx = jnp.full((16, 1), i, jnp.int32)
    dn = lax.GatherDimensionNumbers(offset_dims=(), collapsed_slice_dims=(0,),
                                    start_index_map=(0,))
    return lax.gather(vec, idx, dn, slice_sizes=(1,),
                      mode=lax.GatherScatterMode.PROMISE_IN_BOUNDS)


def _sc_compiler_params():
    return pltpu.CompilerParams(needs_layout_passes=False,
                                use_tc_tiling_on_sc=False)


def _mesh():
    return plsc.VectorSubcoreMesh(core_axis_name="c", subcore_axis_name="s",
                                  num_cores=NC, num_subcores=NT)


def _zero_acc(zbuf, acc, t):
    for i in range(ZR):
        zbuf[i, :] = jnp.zeros((HL,), jnp.float32)

    @pl.loop(0, RPT // ZR)
    def _(k):
        pltpu.sync_copy(zbuf, acc.at[pl.ds(t * RPT + k * ZR, ZR)])


def _deg_body(dst_hbm, ew_hbm, deg_hbm, dstv, ewv, valsv, zbuf, acc, isem, ssem):
    c = lax.axis_index("c")
    t = lax.axis_index("s")
    _zero_acc(zbuf, acc, t)
    plsc.subcore_barrier()

    iota = lax.iota(jnp.int32, HL)
    onehots = [jnp.where(iota == i, 1.0, 0.0).astype(jnp.float32) for i in range(HL)]
    w = c * NT + t
    base = w * DCPT
    nm = jnp.where(w < 31, DMPT0, DMPT1)

    def fetch(m, mb):
        ck = base + m * M
        pltpu.async_copy(dst_hbm.at[pl.ds(ck, M)], dstv.at[mb], isem)
        pltpu.async_copy(ew_hbm.at[pl.ds(ck, M)], ewv.at[mb], isem)

    def fetch_wait(m, mb):
        ck = base + m * M
        pltpu.make_async_copy(dst_hbm.at[pl.ds(ck, M)], dstv.at[mb], isem).wait()
        pltpu.make_async_copy(ew_hbm.at[pl.ds(ck, M)], ewv.at[mb], isem).wait()

    fetch(0, 0)

    @pl.loop(0, nm)
    def _(m):
        mb = lax.rem(m, 2)
        pb = lax.rem(m + 1, 2)
        fetch_wait(m, mb)

        @pl.loop(0, M)
        def _(j):
            @pl.when(m > 0)
            def _():
                pltpu.make_async_copy(valsv.at[pl.ds(j * CK, CK)],
                                      acc.at[dstv.at[pb].at[j]], ssem).wait()

        @pl.when(m + 1 < nm)
        def _():
            fetch(m + 1, pb)

        @pl.loop(0, M)
        def _(j):
            for g in range(8):
                ew16 = ewv[mb, j, pl.ds(g * 16, 16)]
                for i in range(16):
                    valsv[j * CK + g * 16 + i, :] = ew16 * onehots[i]
            pltpu.async_copy(valsv.at[pl.ds(j * CK, CK)],
                             acc.at[dstv.at[mb].at[j]], ssem, add=True)

    lb = lax.rem(nm - 1, 2)

    @pl.loop(0, M)
    def _(j):
        pltpu.make_async_copy(valsv.at[pl.ds(j * CK, CK)],
                              acc.at[dstv.at[lb].at[j]], ssem).wait()
    plsc.subcore_barrier()
    pltpu.sync_copy(acc.at[pl.ds(t * RPT, RPT)],
                    deg_hbm.at[c].at[pl.ds(t * RPT, RPT)])


def _sc_deg(dst2, ew2):
    body = functools.partial(
        pl.kernel,
        out_type=jax.ShapeDtypeStruct((NC, NPAD, HL), jnp.float32),
        mesh=_mesh(),
        compiler_params=_sc_compiler_params(),
        scratch_types=[
            pltpu.VMEM((2, M, CK), jnp.int32),
            pltpu.VMEM((2, M, CK), jnp.float32),
            pltpu.VMEM((M * CK, HL), jnp.float32),
            pltpu.VMEM((ZR, HL), jnp.float32),
            pltpu.VMEM_SHARED((NPAD, HL), jnp.float32),
            pltpu.SemaphoreType.DMA,
            pltpu.SemaphoreType.DMA,
        ],
    )(_deg_body)
    return body(dst2, ew2)


def _agg_body(src_hbm, dst_hbm, ew_hbm, tab_hbm, out_hbm,
              srcv, dstv, ewv, rowsv, zbuf, acc, isem, gsem, ssem):
    c = lax.axis_index("c")
    t = lax.axis_index("s")
    _zero_acc(zbuf, acc, t)
    plsc.subcore_barrier()

    base = t * CPT
    nm = jnp.where(t < 15, MPT0, MPT1)

    def fetch(m, mb):
        ck = base + m * M
        pltpu.async_copy(src_hbm.at[c].at[pl.ds(ck, M)], srcv.at[mb], isem)
        pltpu.async_copy(dst_hbm.at[pl.ds(ck, M)], dstv.at[mb], isem)
        pltpu.async_copy(ew_hbm.at[pl.ds(ck, M)], ewv.at[mb], isem)

    def fetch_wait(m, mb):
        ck = base + m * M
        pltpu.make_async_copy(src_hbm.at[c].at[pl.ds(ck, M)], srcv.at[mb], isem).wait()
        pltpu.make_async_copy(dst_hbm.at[pl.ds(ck, M)], dstv.at[mb], isem).wait()
        pltpu.make_async_copy(ew_hbm.at[pl.ds(ck, M)], ewv.at[mb], isem).wait()

    fetch(0, 0)

    @pl.loop(0, nm)
    def _(m):
        mb = lax.rem(m, 2)
        pb = lax.rem(m + 1, 2)
        fetch_wait(m, mb)
        @pl.loop(0, M)
        def _(j):
            @pl.when(m > 0)
            def _():
                pltpu.make_async_copy(rowsv.at[pl.ds(j * CK, CK)],
                                      acc.at[dstv.at[pb].at[j]], ssem).wait()
            pltpu.async_copy(tab_hbm.at[srcv.at[mb].at[j]],
                             rowsv.at[pl.ds(j * CK, CK)], gsem)

        @pl.when(m + 1 < nm)
        def _():
            fetch(m + 1, pb)

        @pl.loop(0, M)
        def _(j):
            pltpu.make_async_copy(tab_hbm.at[srcv.at[mb].at[j]],
                                  rowsv.at[pl.ds(j * CK, CK)], gsem).wait()
            for g in range(8):
                ew16 = ewv[mb, j, pl.ds(g * 16, 16)]
                for i in range(16):
                    e = j * CK + g * 16 + i
                    b = _bcast_lane(ew16, i)
                    rowsv[e, :] = rowsv[e, :] * b
            pltpu.async_copy(rowsv.at[pl.ds(j * CK, CK)],
                             acc.at[dstv.at[mb].at[j]], ssem, add=True)

    lb = lax.rem(nm - 1, 2)

    @pl.loop(0, M)
    def _(j):
        pltpu.make_async_copy(rowsv.at[pl.ds(j * CK, CK)],
                              acc.at[dstv.at[lb].at[j]], ssem).wait()
    plsc.subcore_barrier()
    pltpu.sync_copy(acc.at[pl.ds(t * RPT, RPT)],
                    out_hbm.at[c].at[pl.ds(t * RPT, RPT)])


def _sc_agg(src2, dst2, ew2, tab2):
    body = functools.partial(
        pl.kernel,
        out_type=jax.ShapeDtypeStruct((NC, NPAD, HL), jnp.float32),
        mesh=_mesh(),
        compiler_params=_sc_compiler_params(),
        scratch_types=[
            pltpu.VMEM((2, M, CK), jnp.int32),
            pltpu.VMEM((2, M, CK), jnp.int32),
            pltpu.VMEM((2, M, CK), jnp.float32),
            pltpu.VMEM((M * CK, HL), jnp.float32),
            pltpu.VMEM((ZR, HL), jnp.float32),
            pltpu.VMEM_SHARED((NPAD, HL), jnp.float32),
            pltpu.SemaphoreType.DMA,
            pltpu.SemaphoreType.DMA,
            pltpu.SemaphoreType.DMA,
        ],
    )(_agg_body)
    return body(src2, dst2, ew2, tab2)


PKR = NPAD * HL // 128
PKH = ROWS * HL // 128


def _pre_kernel(deg3_ref, x_ref, K_ref, D_ref, dis_ref, hs_ref):
    d = deg3_ref[0] + deg3_ref[1]
    deg = jnp.dot(d, K_ref[...], preferred_element_type=jnp.float32) + 1.0
    dis = jnp.where(deg > 0, lax.rsqrt(deg), 0.0)
    dis_ref[...] = dis
    x0 = x_ref[0]
    x1 = x_ref[1]
    for cc in range(NC):
        h = (jnp.dot(x0, D_ref[0, cc], preferred_element_type=jnp.float32)
             + jnp.dot(x1, D_ref[1, cc], preferred_element_type=jnp.float32))
        hs_ref[cc] = h * dis


def _tc_pre(deg3p, xp, K, D):
    return pl.pallas_call(
        _pre_kernel,
        grid=(NBLK,),
        in_specs=[
            pl.BlockSpec((NC, PKH, 128), lambda i: (0, i, 0)),
            pl.BlockSpec((NC, PKH, 128), lambda i: (0, i, 0)),
            pl.BlockSpec((128, 128), lambda i: (0, 0)),
            pl.BlockSpec((NC, NC, 128, 128), lambda i: (0, 0, 0, 0)),
        ],
        out_specs=[
            pl.BlockSpec((PKH, 128), lambda i: (i, 0)),
            pl.BlockSpec((NC, PKH, 128), lambda i: (0, i, 0)),
        ],
        out_shape=[
            jax.ShapeDtypeStruct((PKR, 128), jnp.float32),
            jax.ShapeDtypeStruct((NC, PKR, 128), jnp.float32),
        ],
    )(deg3p, xp, K, D)


def _mid_kernel(acc3_ref, hs3_ref, dis_ref, D_ref, b_ref, out_ref):
    dis = dis_ref[...]
    x0 = dis * (acc3_ref[0] + hs3_ref[0]) + b_ref[0]
    x1 = dis * (acc3_ref[1] + hs3_ref[1]) + b_ref[1]
    for cc in range(NC):
        h = (jnp.dot(x0, D_ref[0, cc], preferred_element_type=jnp.float32)
             + jnp.dot(x1, D_ref[1, cc], preferred_element_type=jnp.float32))
        out_ref[cc] = h * dis


def _tc_mid(acc3p, hs3p, disp, D, bpk):
    return pl.pallas_call(
        _mid_kernel,
        grid=(NBLK,),
        in_specs=[
            pl.BlockSpec((NC, PKH, 128), lambda i: (0, i, 0)),
            pl.BlockSpec((NC, PKH, 128), lambda i: (0, i, 0)),
            pl.BlockSpec((PKH, 128), lambda i: (i, 0)),
            pl.BlockSpec((NC, NC, 128, 128), lambda i: (0, 0, 0, 0)),
            pl.BlockSpec((NC, 1, 128), lambda i: (0, 0, 0)),
        ],
        out_specs=pl.BlockSpec((NC, PKH, 128), lambda i: (0, i, 0)),
        out_shape=jax.ShapeDtypeStruct((NC, PKR, 128), jnp.float32),
    )(acc3p, hs3p, disp, D, bpk)


def _head_kernel(acc3_ref, hs3_ref, dis_ref, b_ref, batch_ref, pW_ref, pb_ref,
                 f1W_ref, f1b_ref, f2W_ref, f2b_ref, oW_ref, ob_ref,
                 out_ref, acc_ref):
    i = pl.program_id(0)

    @pl.when(i == 0)
    def _():
        acc_ref[...] = jnp.zeros_like(acc_ref)

    dis = dis_ref[...]
    x0 = jnp.maximum(dis * (acc3_ref[0] + hs3_ref[0]) + b_ref[0], 0.0)
    x1 = jnp.maximum(dis * (acc3_ref[1] + hs3_ref[1]) + b_ref[1], 0.0)
    lane = lax.broadcasted_iota(jnp.int32, (PKH, 128), 1)
    x1 = jnp.where(lax.rem(lane, 16) == 15, 1.0, x1)
    gid = lax.broadcasted_iota(jnp.int32, (G, PKH), 0)
    pooled = jnp.zeros((G, DP), jnp.float32)
    for k in range(8):
        bk = batch_ref[0, k]
        mk = (gid == bk[None, :]).astype(jnp.float32)
        s0 = jnp.dot(mk, x0[:, 16 * k:16 * k + 16],
                     preferred_element_type=jnp.float32)
        s1 = jnp.dot(mk, x1[:, 16 * k:16 * k + 16],
                     preferred_element_type=jnp.float32)
        pooled += jnp.concatenate([s0, s1], axis=1)
    acc_ref[...] += pooled

    @pl.when(i == NBLK - 1)
    def _():
        acc = acc_ref[...]
        cnt = jnp.maximum(acc[:, DP - 1], 1.0)
        mean = acc / cnt[:, None]

        def leaky(v):
            return jnp.where(v > 0, v, 0.01 * v)
        z = leaky(jnp.dot(mean, pW_ref[...], preferred_element_type=jnp.float32)
                  + pb_ref[...])
        z = leaky(jnp.dot(z, f1W_ref[...], preferred_element_type=jnp.float32)
                  + f1b_ref[...])
        z = leaky(jnp.dot(z, f2W_ref[...], preferred_element_type=jnp.float32)
                  + f2b_ref[...])
        logits = jnp.dot(z, oW_ref[...], preferred_element_type=jnp.float32) + ob_ref[...]
        lane2 = lax.broadcasted_iota(jnp.int32, logits.shape, 1)
        valid = lane2 < 5
        m = jnp.max(jnp.where(valid, logits, -jnp.inf), axis=1, keepdims=True)
        e = jnp.where(valid, jnp.exp(logits - m), 0.0)
        out_ref[...] = e / jnp.sum(e, axis=1, keepdims=True)


def _tc_head(acc3p, hs3p, disp, b3pk, batchS, pWp, pbp, f1W, f1bp, f2Wp, f2bp,
             oWp, obp):
    full = lambda shape: pl.BlockSpec(shape, lambda i: (0,) * len(shape))
    return pl.pallas_call(
        _head_kernel,
        grid=(NBLK,),
        in_specs=[
            pl.BlockSpec((NC, PKH, 128), lambda i: (0, i, 0)),
            pl.BlockSpec((NC, PKH, 128), lambda i: (0, i, 0)),
            pl.BlockSpec((PKH, 128), lambda i: (i, 0)),
            full((NC, 1, 128)),
            pl.BlockSpec((1, 8, PKH), lambda i: (i, 0, 0)),
            full((DP, FP)), full((1, FP)),
            full((FP, 256)), full((1, 256)),
            full((256, FP)), full((1, FP)),
            full((FP, FP)), full((1, FP)),
        ],
        out_specs=pl.BlockSpec((G, FP), lambda i: (0, 0)),
        out_shape=jax.ShapeDtypeStruct((G, FP), jnp.float32),
        scratch_shapes=[pltpu.VMEM((G, DP), jnp.float32)],
    )(acc3p, hs3p, disp, b3pk, batchS, pWp, pbp, f1W, f1bp, f2Wp, f2bp, oWp, obp)


def _blockdiag(Wp):
    eye8 = jnp.eye(8, dtype=jnp.float32)
    blocks = [[jnp.kron(eye8, Wp[16 * c:16 * c + 16, 16 * cc:16 * cc + 16])
               for cc in range(NC)] for c in range(NC)]
    return jnp.stack([jnp.stack(r) for r in blocks])


def _bias_pk(bp):
    return jnp.stack([jnp.tile(bp[0, 16 * c:16 * c + 16], 8)[None, :]
                      for c in range(NC)])


def kernel(prot_x, prot_edge_index, prot_dist, prot_batch, W1, b1, W2, b2, W3,
           b3, pW, pb, f1W, f1b, f2W, f2b, oW, ob):
    src2 = prot_edge_index[0].reshape(CHUNKS, CK)
    srcs2 = jnp.stack([src2, src2 + NPAD])
    dst2 = prot_edge_index[1].reshape(CHUNKS, CK)
    ew2 = prot_dist.reshape(CHUNKS, CK)

    x32 = jnp.pad(prot_x, ((0, NPAD - N), (0, DP - F)))
    xp = jnp.stack([x32[:, :HL], x32[:, HL:]]).reshape(NC, PKR, 128)
    K = jnp.kron(jnp.eye(8, dtype=jnp.float32), jnp.ones((HL, HL), jnp.float32))
    W1p = jnp.pad(W1, ((0, DP - F), (0, DP - F)))
    W2p = jnp.pad(W2, ((0, DP - F), (0, DP - F)))
    W3p = jnp.pad(W3, ((0, DP - F), (0, DP - F)))
    D1, D2, D3 = _blockdiag(W1p), _blockdiag(W2p), _blockdiag(W3p)
    b1k = _bias_pk(jnp.pad(b1, (0, DP - F))[None, :])
    b2k = _bias_pk(jnp.pad(b2, (0, DP - F))[None, :])
    b3k = _bias_pk(jnp.pad(b3, (0, DP - F))[None, :])

    pk = lambda a3: a3.reshape(NC, PKR, 128)
    tab = lambda h: h.reshape(NC * NPAD, HL)
    deg3 = _sc_deg(dst2, ew2)
    disp, hs3p = _tc_pre(pk(deg3), xp, K, D1)
    acc3 = _sc_agg(srcs2, dst2, ew2, tab(hs3p))
    hs3p = _tc_mid(pk(acc3), hs3p, disp, D2, b1k)
    acc3 = _sc_agg(srcs2, dst2, ew2, tab(hs3p))
    hs3p = _tc_mid(pk(acc3), hs3p, disp, D3, b2k)
    acc3 = _sc_agg(srcs2, dst2, ew2, tab(hs3p))

    batchS = jnp.pad(prot_batch, (0, NPAD - N), constant_values=G) \
        .reshape(NBLK, PKH, 8).transpose(0, 2, 1)
    pWp = jnp.pad(pW, ((0, DP - F), (0, 0)))
    pbp = pb[None, :]
    f1bp = f1b[None, :]
    f2Wp = jnp.pad(f2W, ((0, 0), (0, FP - 64)))
    f2bp = jnp.pad(f2b, (0, FP - 64))[None, :]
    oWp = jnp.pad(oW, ((0, FP - 64), (0, FP - 5)))
    obp = jnp.pad(ob, (0, FP - 5))[None, :]
    out = _tc_head(pk(acc3), hs3p, disp, b3k, batchS, pWp, pbp, f1W, f1bp,
                   f2Wp, f2bp, oWp, obp)
    return out[:, :5]
